# baseline (speedup 1.0000x reference)
import functools

import jax
import jax.numpy as jnp
from jax import lax
from jax.experimental import pallas as pl
from jax.experimental.pallas import tpu as pltpu
from jax.experimental.pallas import tpu_sc as plsc

EMB = 2048
NE = 16
NTOK = 16384
TC_BLK = 2048
SUB = TC_BLK // 512
NW = 32
TOK_PER_W = NTOK // NW
LANES = 16
GROUPS = TOK_PER_W // LANES


def _tc_scores_body(x_ref, w_ref, b_ref, out_ref):
    for j in range(SUB):
        s = lax.dot_general(
            w_ref[...], x_ref[pl.ds(j * 512, 512), :],
            (((1,), (1,)), ((), ())),
            preferred_element_type=jnp.float32,
            precision=lax.Precision.DEFAULT,
        )
        out_ref[j] = s + b_ref[...]


def _tc_scores(x, W, b2):
    return pl.pallas_call(
        _tc_scores_body,
        grid=(NTOK // TC_BLK,),
        in_specs=[
            pl.BlockSpec((TC_BLK, EMB), lambda i: (i, 0)),
            pl.BlockSpec((NE, EMB), lambda i: (0, 0)),
            pl.BlockSpec((NE, 1), lambda i: (0, 0)),
        ],
        out_specs=pl.BlockSpec((SUB, NE, 512), lambda i: (i, 0, 0)),
        out_shape=jax.ShapeDtypeStruct((NW, NE, TOK_PER_W), jnp.float32),
        compiler_params=pltpu.CompilerParams(
            dimension_semantics=("arbitrary",)),
    )(x, W, b2)


def _sc_topk(scores):
    mesh = plsc.VectorSubcoreMesh(core_axis_name="c", subcore_axis_name="s")

    @functools.partial(
        pl.kernel,
        mesh=mesh,
        out_type=jax.ShapeDtypeStruct((NW, 4, TOK_PER_W), jnp.float32),
        scratch_types=[
            pltpu.VMEM((NE, TOK_PER_W), jnp.float32),
            pltpu.VMEM((4, TOK_PER_W), jnp.float32),
        ],
    )
    def k(scores_hbm, out_hbm, sc_v, out_v):
        wid = lax.axis_index("s") * 2 + lax.axis_index("c")
        pltpu.sync_copy(scores_hbm.at[wid], sc_v)

        for g in range(GROUPS):
            col = pl.ds(g * LANES, LANES)
            m = [sc_v[e, col] for e in range(NE)]
            i = [jnp.full((LANES,), e, jnp.int32) for e in range(NE)]
            while len(m) > 1:
                nm, ni = [], []
                for a in range(0, len(m), 2):
                    gt = m[a + 1] > m[a]
                    nm.append(jnp.where(gt, m[a + 1], m[a]))
                    ni.append(jnp.where(gt, i[a + 1], i[a]))
                m, i = nm, ni
            m1, i1 = m[0], i[0]
            neg = jnp.float32(-jnp.inf)
            m = [jnp.where(i1 == e, neg, sc_v[e, col]) for e in range(NE)]
            i = [jnp.full((LANES,), e, jnp.int32) for e in range(NE)]
            while len(m) > 1:
                nm, ni = [], []
                for a in range(0, len(m), 2):
                    gt = m[a + 1] > m[a]
                    nm.append(jnp.where(gt, m[a + 1], m[a]))
                    ni.append(jnp.where(gt, i[a + 1], i[a]))
                m, i = nm, ni
            m2, i2 = m[0], i[0]
            e2 = jnp.exp(m2 - m1)
            den = e2 + jnp.float32(1.0)
            out_v[0, col] = jnp.float32(1.0) / den
            out_v[1, col] = e2 / den
            out_v[2, col] = lax.bitcast_convert_type(i1, jnp.float32)
            out_v[3, col] = lax.bitcast_convert_type(i2, jnp.float32)

        pltpu.sync_copy(out_v, out_hbm.at[wid])

    return k(scores)


def kernel(x, W, b):
    scores = _tc_scores(x, W, jnp.reshape(b, (NE, 1)))
    out = _sc_topk(scores)
    v1 = out[:, 0, :].reshape(NTOK)
    v2 = out[:, 1, :].reshape(NTOK)
    i1 = out[:, 2, :].reshape(NTOK).view(jnp.int32)
    i2 = out[:, 3, :].reshape(NTOK).view(jnp.int32)
    values = jnp.stack([v1, v2], axis=-1)
    indices = jnp.stack([i1, i2], axis=-1)
    return values, indices


# speedup vs baseline: 1.0166x; 1.0166x over previous
import functools

import jax
import jax.numpy as jnp
from jax import lax
from jax.experimental import pallas as pl
from jax.experimental.pallas import tpu as pltpu
from jax.experimental.pallas import tpu_sc as plsc

EMB = 2048
NE = 16
NTOK = 16384
TC_BLK = 2048
SUB = TC_BLK // 512
NW = 32
TOK_PER_W = NTOK // NW
LANES = 16
GROUPS = TOK_PER_W // LANES


def _tc_scores_body(x_ref, w_ref, b_ref, out_ref):
    for j in range(SUB):
        s = lax.dot_general(
            w_ref[...], x_ref[pl.ds(j * 512, 512), :],
            (((1,), (1,)), ((), ())),
            preferred_element_type=jnp.float32,
            precision=lax.Precision.DEFAULT,
        )
        out_ref[j] = s + b_ref[...]


def _tc_scores(x, W, b2):
    return pl.pallas_call(
        _tc_scores_body,
        grid=(NTOK // TC_BLK,),
        in_specs=[
            pl.BlockSpec((TC_BLK, EMB), lambda i: (i, 0)),
            pl.BlockSpec((NE, EMB), lambda i: (0, 0)),
            pl.BlockSpec((NE, 1), lambda i: (0, 0)),
        ],
        out_specs=pl.BlockSpec((SUB, NE, 512), lambda i: (i, 0, 0)),
        out_shape=jax.ShapeDtypeStruct((NW, NE, TOK_PER_W), jnp.float32),
        compiler_params=pltpu.CompilerParams(
            dimension_semantics=("arbitrary",)),
    )(x, W, b2)


def _sc_topk(scores):
    mesh = plsc.VectorSubcoreMesh(core_axis_name="c", subcore_axis_name="s")

    @functools.partial(
        pl.kernel,
        mesh=mesh,
        out_type=jax.ShapeDtypeStruct((NW, 4, TOK_PER_W), jnp.float32),
        scratch_types=[
            pltpu.VMEM((NE, TOK_PER_W), jnp.float32),
            pltpu.VMEM((4, TOK_PER_W), jnp.float32),
        ],
    )
    def k(scores_hbm, out_hbm, sc_v, out_v):
        wid = lax.axis_index("s") * 2 + lax.axis_index("c")
        pltpu.sync_copy(scores_hbm.at[wid], sc_v)

        def group(g, carry):
            col = pl.ds(pl.multiple_of(g * LANES, LANES), LANES)
            m = [sc_v[e, col] for e in range(NE)]
            i = [jnp.full((LANES,), e, jnp.int32) for e in range(NE)]
            while len(m) > 1:
                nm, ni = [], []
                for a in range(0, len(m), 2):
                    gt = m[a + 1] > m[a]
                    nm.append(jnp.where(gt, m[a + 1], m[a]))
                    ni.append(jnp.where(gt, i[a + 1], i[a]))
                m, i = nm, ni
            m1, i1 = m[0], i[0]
            neg = jnp.float32(-jnp.inf)
            m = [jnp.where(i1 == e, neg, sc_v[e, col]) for e in range(NE)]
            i = [jnp.full((LANES,), e, jnp.int32) for e in range(NE)]
            while len(m) > 1:
                nm, ni = [], []
                for a in range(0, len(m), 2):
                    gt = m[a + 1] > m[a]
                    nm.append(jnp.where(gt, m[a + 1], m[a]))
                    ni.append(jnp.where(gt, i[a + 1], i[a]))
                m, i = nm, ni
            m2, i2 = m[0], i[0]
            e2 = jnp.exp(m2 - m1)
            den = e2 + jnp.float32(1.0)
            out_v[0, col] = jnp.float32(1.0) / den
            out_v[1, col] = e2 / den
            out_v[2, col] = lax.bitcast_convert_type(i1, jnp.float32)
            out_v[3, col] = lax.bitcast_convert_type(i2, jnp.float32)
            return carry

        lax.fori_loop(0, GROUPS, group, 0)
        pltpu.sync_copy(out_v, out_hbm.at[wid])

    return k(scores)


def kernel(x, W, b):
    scores = _tc_scores(x, W, jnp.reshape(b, (NE, 1)))
    out = _sc_topk(scores)
    v1 = out[:, 0, :].reshape(NTOK)
    v2 = out[:, 1, :].reshape(NTOK)
    i1 = out[:, 2, :].reshape(NTOK).view(jnp.int32)
    i2 = out[:, 3, :].reshape(NTOK).view(jnp.int32)
    values = jnp.stack([v1, v2], axis=-1)
    indices = jnp.stack([i1, i2], axis=-1)
    return values, indices


# speedup vs baseline: 1.0208x; 1.0041x over previous
import functools

import jax
import jax.numpy as jnp
from jax import lax
from jax.experimental import pallas as pl
from jax.experimental.pallas import tpu as pltpu
from jax.experimental.pallas import tpu_sc as plsc

EMB = 2048
NE = 16
NTOK = 16384
TC_BLK = 2048
SUB = TC_BLK // 512
NW = 32
TOK_PER_W = NTOK // NW
LANES = 16
GROUPS = TOK_PER_W // LANES


def _tc_scores_body(x_ref, w_ref, b_ref, out_ref):
    for j in range(SUB):
        s = lax.dot_general(
            w_ref[...], x_ref[pl.ds(j * 512, 512), :],
            (((1,), (1,)), ((), ())),
            preferred_element_type=jnp.float32,
            precision=lax.Precision.DEFAULT,
        )
        out_ref[j] = s + b_ref[...]


def _tc_scores(x, W, b2):
    return pl.pallas_call(
        _tc_scores_body,
        grid=(NTOK // TC_BLK,),
        in_specs=[
            pl.BlockSpec((TC_BLK, EMB), lambda i: (i, 0)),
            pl.BlockSpec((NE, EMB), lambda i: (0, 0)),
            pl.BlockSpec((NE, 1), lambda i: (0, 0)),
        ],
        out_specs=pl.BlockSpec((SUB, NE, 512), lambda i: (i, 0, 0)),
        out_shape=jax.ShapeDtypeStruct((NW, NE, TOK_PER_W), jnp.float32),
        compiler_params=pltpu.CompilerParams(
            dimension_semantics=("parallel",)),
    )(x, W, b2)


def _sc_topk(scores):
    mesh = plsc.VectorSubcoreMesh(core_axis_name="c", subcore_axis_name="s")

    @functools.partial(
        pl.kernel,
        mesh=mesh,
        out_type=jax.ShapeDtypeStruct((NW, 4, TOK_PER_W), jnp.float32),
        scratch_types=[
            pltpu.VMEM((NE, TOK_PER_W), jnp.float32),
            pltpu.VMEM((4, TOK_PER_W), jnp.float32),
        ],
    )
    def k(scores_hbm, out_hbm, sc_v, out_v):
        wid = lax.axis_index("s") * 2 + lax.axis_index("c")
        pltpu.sync_copy(scores_hbm.at[wid], sc_v)

        def group(g, carry):
            col = pl.ds(pl.multiple_of(g * LANES, LANES), LANES)
            m = [sc_v[e, col] for e in range(NE)]
            i = [jnp.full((LANES,), e, jnp.int32) for e in range(NE)]
            while len(m) > 1:
                nm, ni = [], []
                for a in range(0, len(m), 2):
                    gt = m[a + 1] > m[a]
                    nm.append(jnp.where(gt, m[a + 1], m[a]))
                    ni.append(jnp.where(gt, i[a + 1], i[a]))
                m, i = nm, ni
            m1, i1 = m[0], i[0]
            neg = jnp.float32(-jnp.inf)
            m = [jnp.where(i1 == e, neg, sc_v[e, col]) for e in range(NE)]
            i = [jnp.full((LANES,), e, jnp.int32) for e in range(NE)]
            while len(m) > 1:
                nm, ni = [], []
                for a in range(0, len(m), 2):
                    gt = m[a + 1] > m[a]
                    nm.append(jnp.where(gt, m[a + 1], m[a]))
                    ni.append(jnp.where(gt, i[a + 1], i[a]))
                m, i = nm, ni
            m2, i2 = m[0], i[0]
            e2 = jnp.exp(m2 - m1)
            den = e2 + jnp.float32(1.0)
            out_v[0, col] = jnp.float32(1.0) / den
            out_v[1, col] = e2 / den
            out_v[2, col] = lax.bitcast_convert_type(i1, jnp.float32)
            out_v[3, col] = lax.bitcast_convert_type(i2, jnp.float32)
            return carry

        lax.fori_loop(0, GROUPS, group, 0)
        pltpu.sync_copy(out_v, out_hbm.at[wid])

    return k(scores)


def kernel(x, W, b):
    scores = _tc_scores(x, W, jnp.reshape(b, (NE, 1)))
    out = _sc_topk(scores)
    v1 = out[:, 0, :].reshape(NTOK)
    v2 = out[:, 1, :].reshape(NTOK)
    i1 = out[:, 2, :].reshape(NTOK).view(jnp.int32)
    i2 = out[:, 3, :].reshape(NTOK).view(jnp.int32)
    values = jnp.stack([v1, v2], axis=-1)
    indices = jnp.stack([i1, i2], axis=-1)
    return values, indices


# speedup vs baseline: 1.0261x; 1.0052x over previous
import functools

import jax
import jax.numpy as jnp
from jax import lax
from jax.experimental import pallas as pl
from jax.experimental.pallas import tpu as pltpu
from jax.experimental.pallas import tpu_sc as plsc

EMB = 2048
NE = 16
NTOK = 16384
TC_BLK = 2048
SUB = TC_BLK // 512
NW = 32
TOK_PER_W = NTOK // NW
LANES = 16
GROUPS = TOK_PER_W // LANES


def _tc_scores_body(x_ref, w_ref, b_ref, out_ref):
    for j in range(SUB):
        s = lax.dot_general(
            w_ref[...], x_ref[pl.ds(j * 512, 512), :],
            (((1,), (1,)), ((), ())),
            preferred_element_type=jnp.float32,
            precision=lax.Precision.DEFAULT,
        )
        out_ref[j] = s + b_ref[...]


def _tc_scores(x, W, b2):
    return pl.pallas_call(
        _tc_scores_body,
        grid=(NTOK // TC_BLK,),
        in_specs=[
            pl.BlockSpec((TC_BLK, EMB), lambda i: (i, 0)),
            pl.BlockSpec((NE, EMB), lambda i: (0, 0)),
            pl.BlockSpec((NE, 1), lambda i: (0, 0)),
        ],
        out_specs=pl.BlockSpec((SUB, NE, 512), lambda i: (i, 0, 0)),
        out_shape=jax.ShapeDtypeStruct((NW, NE, TOK_PER_W), jnp.float32),
        compiler_params=pltpu.CompilerParams(
            dimension_semantics=("parallel",)),
    )(x, W, b2)


def _sc_topk(scores):
    mesh = plsc.VectorSubcoreMesh(core_axis_name="c", subcore_axis_name="s")

    @functools.partial(
        pl.kernel,
        mesh=mesh,
        out_type=jax.ShapeDtypeStruct((NW, 4, TOK_PER_W), jnp.float32),
        scratch_types=[
            pltpu.VMEM((NE, TOK_PER_W), jnp.float32),
            pltpu.VMEM((4, TOK_PER_W), jnp.float32),
        ],
    )
    def k(scores_hbm, out_hbm, sc_v, out_v):
        wid = lax.axis_index("s") * 2 + lax.axis_index("c")
        pltpu.sync_copy(scores_hbm.at[wid], sc_v)

        def tournament(m, i):
            while len(m) > 1:
                nm, ni = [], []
                for a in range(0, len(m), 2):
                    gt = m[a + 1] > m[a]
                    nm.append(jnp.where(gt, m[a + 1], m[a]))
                    ni.append(jnp.where(gt, i[a + 1], i[a]))
                m, i = nm, ni
            return m[0], i[0]

        @plsc.parallel_loop(0, GROUPS, 1, unroll=2)
        def group(g):
            col = pl.ds(pl.multiple_of(g * LANES, LANES), LANES)
            regs = [sc_v[e, col] for e in range(NE)]
            idxs = [jnp.full((LANES,), e, jnp.int32) for e in range(NE)]
            m1, i1 = tournament(list(regs), list(idxs))
            neg = jnp.float32(-jnp.inf)
            masked = [jnp.where(i1 == e, neg, regs[e]) for e in range(NE)]
            m2, i2 = tournament(masked, list(idxs))
            e2 = jnp.exp(m2 - m1)
            den = e2 + jnp.float32(1.0)
            out_v[0, col] = jnp.float32(1.0) / den
            out_v[1, col] = e2 / den
            out_v[2, col] = lax.bitcast_convert_type(i1, jnp.float32)
            out_v[3, col] = lax.bitcast_convert_type(i2, jnp.float32)
        pltpu.sync_copy(out_v, out_hbm.at[wid])

    return k(scores)


def kernel(x, W, b):
    scores = _tc_scores(x, W, jnp.reshape(b, (NE, 1)))
    out = _sc_topk(scores)
    v1 = out[:, 0, :].reshape(NTOK)
    v2 = out[:, 1, :].reshape(NTOK)
    i1 = out[:, 2, :].reshape(NTOK).view(jnp.int32)
    i2 = out[:, 3, :].reshape(NTOK).view(jnp.int32)
    values = jnp.stack([v1, v2], axis=-1)
    indices = jnp.stack([i1, i2], axis=-1)
    return values, indices
